# Initial kernel scaffold; baseline (speedup 1.0000x reference)
import jax
import jax.numpy as jnp
from jax.experimental import pallas as pl


def kernel(m_data, Wv, bv, feat_table, univ_table, flag_table):
    raise NotImplementedError("write your pallas kernel here")


# speedup vs baseline: 12.4264x; 12.4264x over previous
import jax
import jax.numpy as jnp
from jax import lax
from jax.experimental import pallas as pl
from jax.experimental.pallas import tpu as pltpu
from jax.experimental.pallas import tpu_sc as plsc

U, S, F, D = 33, 256, 64, 64
L = 16
NJ = D // L
NC, NS = 2, 16
NW = NC * NS
ROWS = U * S * F
CHUNK = 512
SPC = CHUNK // F
TASKS = ROWS // CHUNK
TPW = TASKS // NW
CPU_ = S * F // CHUNK
MVALS = TPW * CHUNK


def _sc_body(mf_hbm, wv_hbm, bv_hbm, feat_hbm, univ_hbm, flag_hbm, out_hbm,
             m_v, feat_v, univ_v, flag_v, bv_v, wv_v, base_v,
             obuf0, obuf1, sem0, sem1):
    w = lax.axis_index("s") * NC + lax.axis_index("c")

    pltpu.sync_copy(mf_hbm.at[pl.ds(w * MVALS, MVALS)], m_v)
    pltpu.sync_copy(feat_hbm, feat_v)
    pltpu.sync_copy(univ_hbm, univ_v)
    pltpu.sync_copy(flag_hbm, flag_v)
    pltpu.sync_copy(bv_hbm, bv_v)
    pltpu.sync_copy(wv_hbm, wv_v)

    wvj = [wv_v[pl.ds(L * j, L)] for j in range(NJ)]
    zero16 = jnp.zeros((L,), jnp.float32)

    def compute_base(u):
        upred = jnp.full((L,), u > 0)
        pres, dfls = [], []
        for j in range(NJ):
            uv = jnp.where(upred, univ_v[pl.ds(D + L * j, L)],
                           univ_v[pl.ds(L * j, L)])
            pres.append(bv_v[pl.ds(L * j, L)] + uv + flag_v[pl.ds(L * j, L)])
            dfls.append(flag_v[pl.ds(D + L * j, L)] - flag_v[pl.ds(L * j, L)])

        def fbody(f, c):
            fpred = jnp.full((L,), f == u - 1)
            for j in range(NJ):
                base_v[pl.ds(f * D + L * j, L)] = (
                    pres[j] + feat_v[pl.ds(f * D + L * j, L)]
                    + jnp.where(fpred, dfls[j], zero16))
            return c

        lax.fori_loop(0, F, fbody, 0)

    def fill_buf(ti, buf):
        def fbody(f, c):
            basej = [base_v[pl.ds(f * D + L * j, L)] for j in range(NJ)]
            for s8 in range(SPC):
                midx = jnp.full((L,), (ti * SPC + s8) * F, jnp.int32) + f
                mv = plsc.load_gather(m_v, [midx])
                r = s8 * F + f
                for j in range(NJ):
                    buf[pl.ds(r * D + L * j, L)] = mv * wvj[j] + basej[j]
            return c

        lax.fori_loop(0, F, fbody, 0)

    def do_task(ti, prev_u, buf, sem, wait_first):
        t = w * TPW + ti
        u = t // CPU_

        @pl.when(u != prev_u)
        def _():
            compute_base(u)

        if wait_first:
            t2 = t - 2
            pltpu.make_async_copy(
                buf, out_hbm.at[pl.ds(t2 * CHUNK * D, CHUNK * D)], sem).wait()
        fill_buf(ti, buf)
        pltpu.async_copy(buf, out_hbm.at[pl.ds(t * CHUNK * D, CHUNK * D)], sem)
        return u

    prev_u = do_task(0, jnp.int32(-1), obuf0, sem0, False)
    prev_u = do_task(1, prev_u, obuf1, sem1, False)

    def pair_body(p, pu):
        pu = do_task(2 * p, pu, obuf0, sem0, True)
        pu = do_task(2 * p + 1, pu, obuf1, sem1, True)
        return pu

    prev_u = lax.fori_loop(1, TPW // 2, pair_body, prev_u)
    do_task(TPW - 1, prev_u, obuf0, sem0, True)

    t_a = w * TPW + TPW - 2
    pltpu.make_async_copy(
        obuf1, out_hbm.at[pl.ds(t_a * CHUNK * D, CHUNK * D)], sem1).wait()
    t_b = w * TPW + TPW - 1
    pltpu.make_async_copy(
        obuf0, out_hbm.at[pl.ds(t_b * CHUNK * D, CHUNK * D)], sem0).wait()


@jax.jit
def _sc_embed(mf, wv, bv, feat, univ, flag):
    mesh = plsc.VectorSubcoreMesh(
        core_axis_name="c", subcore_axis_name="s",
        num_cores=NC, num_subcores=NS)
    run = pl.kernel(
        _sc_body,
        out_type=jax.ShapeDtypeStruct((ROWS * D,), jnp.float32),
        mesh=mesh,
        compiler_params=pltpu.CompilerParams(needs_layout_passes=False),
        scratch_types=[
            pltpu.VMEM((MVALS,), jnp.float32),
            pltpu.VMEM((F * D,), jnp.float32),
            pltpu.VMEM((2 * D,), jnp.float32),
            pltpu.VMEM((2 * D,), jnp.float32),
            pltpu.VMEM((D,), jnp.float32),
            pltpu.VMEM((D,), jnp.float32),
            pltpu.VMEM((F * D,), jnp.float32),
            pltpu.VMEM((CHUNK * D,), jnp.float32),
            pltpu.VMEM((CHUNK * D,), jnp.float32),
            pltpu.SemaphoreType.DMA,
            pltpu.SemaphoreType.DMA,
        ],
    )
    return run(mf, wv, bv, feat, univ, flag)


def kernel(m_data, Wv, bv, feat_table, univ_table, flag_table):
    mf = m_data.reshape(U * S * F)
    wv = Wv.reshape(D)
    out = _sc_embed(mf, wv, bv, feat_table.reshape(F * D),
                    univ_table.reshape(2 * D), flag_table.reshape(2 * D))
    return out.reshape(U, S * F, D)
